# Initial kernel scaffold; baseline (speedup 1.0000x reference)
import jax
import jax.numpy as jnp
from jax.experimental import pallas as pl


def kernel(angles, table):
    raise NotImplementedError("write your pallas kernel here")


# speedup vs baseline: 4.2451x; 4.2451x over previous
import functools
import math

import jax
import jax.numpy as jnp
from jax import lax
from jax.experimental import pallas as pl
from jax.experimental.pallas import tpu as pltpu
from jax.experimental.pallas import tpu_sc as plsc

_EMBED = 64
_ROWS = 16384
_COLS = 50
_B = _ROWS * _COLS

_NC = 2
_NS = 16
_NW = _NC * _NS
_BPW = _B // _NW
_CH = 128
_NCHUNK = _BPW // _CH


def _sc_gather_body(ang_hbm, table_hbm, out_hbm, ang_v, idx_v, rows_v, sem):
    wid = lax.axis_index("s") * _NC + lax.axis_index("c")
    base = wid * _BPW

    def chunk(g, carry):
        off = base + g * _CH
        pltpu.sync_copy(ang_hbm.at[pl.ds(off, _CH)], ang_v)
        for i in range(_CH // 16):
            sl = pl.ds(i * 16, 16)
            idx_v[sl] = ang_v[sl].astype(jnp.int32)
        pltpu.async_copy(table_hbm.at[idx_v], rows_v, sem).wait()
        pltpu.sync_copy(rows_v, out_hbm.at[pl.ds(off, _CH)])
        return carry

    lax.fori_loop(0, _NCHUNK, chunk, 0)


_sc_gather = pl.kernel(
    _sc_gather_body,
    out_type=jax.ShapeDtypeStruct((_B, _EMBED), jnp.float32),
    mesh=plsc.VectorSubcoreMesh(core_axis_name="c", subcore_axis_name="s"),
    scratch_types=[
        pltpu.VMEM((_CH,), jnp.float32),
        pltpu.VMEM((_CH,), jnp.int32),
        pltpu.VMEM((_CH, _EMBED), jnp.float32),
        pltpu.SemaphoreType.DMA,
    ],
    compiler_params=pltpu.CompilerParams(use_tc_tiling_on_sc=False),
)


def _tc_trig_body(a_ref, rad_ref, sin_ref, cos_ref):
    r = a_ref[...] * jnp.float32(math.pi / 180.0)
    rad_ref[...] = r
    sin_ref[...] = jnp.sin(r)
    cos_ref[...] = jnp.cos(r)


_TC_BLOCK = 1024

_tc_trig = pl.pallas_call(
    _tc_trig_body,
    grid=(_ROWS // _TC_BLOCK,),
    in_specs=[pl.BlockSpec((_TC_BLOCK, _COLS), lambda i: (i, 0))],
    out_specs=[pl.BlockSpec((_TC_BLOCK, _COLS), lambda i: (i, 0))] * 3,
    out_shape=[jax.ShapeDtypeStruct((_ROWS, _COLS), jnp.float32)] * 3,
)


def kernel(angles, table):
    radians, sin_enc, cos_enc = _tc_trig(angles)
    rows = _sc_gather(angles.reshape(-1), table)
    return radians, sin_enc, cos_enc, rows.reshape(_ROWS, _COLS, _EMBED)


# speedup vs baseline: 4.4354x; 1.0448x over previous
import math

import jax
import jax.numpy as jnp
from jax import lax
from jax.experimental import pallas as pl
from jax.experimental.pallas import tpu as pltpu
from jax.experimental.pallas import tpu_sc as plsc

_EMBED = 64
_ROWS = 16384
_COLS = 50
_B = _ROWS * _COLS

_NC = 2
_NS = 16
_NW = _NC * _NS
_BPW = _B // _NW
_IW = 128
_IROWS = _BPW // _IW
_SC = 512
_GPS = _SC // _IW
_NSC = _BPW // _SC


def _sc_gather_body(ang_hbm, table_hbm, out_hbm,
                    ang_v, idx_v, rows0, rows1, gsem0, gsem1, ssem0, ssem1):
    wid = lax.axis_index("s") * _NC + lax.axis_index("c")
    base_i = wid * _IROWS
    base_r = wid * _BPW
    rows = (rows0, rows1)
    gsems = (gsem0, gsem1)
    ssems = (ssem0, ssem1)

    pltpu.sync_copy(ang_hbm.at[pl.ds(base_i, _IROWS)], ang_v)

    def conv(r, carry):
        for i in range(_IW // 16):
            sl = pl.ds(i * 16, 16)
            idx_v[r, sl] = ang_v[r, sl].astype(jnp.int32)
        return carry

    lax.fori_loop(0, _IROWS, conv, 0)

    def start_gather(s, b):
        for j in range(_GPS):
            pltpu.make_async_copy(
                table_hbm.at[idx_v.at[s * _GPS + j]],
                rows[b].at[pl.ds(j * _IW, _IW)],
                gsems[b],
            ).start()

    def wait_gather(s, b):
        for j in range(_GPS):
            pltpu.make_async_copy(
                table_hbm.at[idx_v.at[s * _GPS + j]],
                rows[b].at[pl.ds(j * _IW, _IW)],
                gsems[b],
            ).wait()

    def scatter_desc(s, b):
        return pltpu.make_async_copy(
            rows[b], out_hbm.at[pl.ds(base_r + s * _SC, _SC)], ssems[b])

    start_gather(0, 0)
    start_gather(1, 1)

    def body(p, carry):
        s0 = 2 * p
        wait_gather(s0, 0)
        scatter_desc(s0, 0).start()
        scatter_desc(s0, 0).wait()
        start_gather(s0 + 2, 0)
        wait_gather(s0 + 1, 1)
        scatter_desc(s0 + 1, 1).start()
        scatter_desc(s0 + 1, 1).wait()
        start_gather(s0 + 3, 1)
        return carry

    lax.fori_loop(0, (_NSC - 2) // 2, body, 0)

    wait_gather(_NSC - 2, 0)
    scatter_desc(_NSC - 2, 0).start()
    wait_gather(_NSC - 1, 1)
    scatter_desc(_NSC - 1, 1).start()
    scatter_desc(_NSC - 2, 0).wait()
    scatter_desc(_NSC - 1, 1).wait()


_sc_gather = pl.kernel(
    _sc_gather_body,
    out_type=jax.ShapeDtypeStruct((_B, _EMBED), jnp.float32),
    mesh=plsc.VectorSubcoreMesh(core_axis_name="c", subcore_axis_name="s"),
    scratch_types=[
        pltpu.VMEM((_IROWS, _IW), jnp.float32),
        pltpu.VMEM((_IROWS, _IW), jnp.int32),
        pltpu.VMEM((_SC, _EMBED), jnp.float32),
        pltpu.VMEM((_SC, _EMBED), jnp.float32),
        pltpu.SemaphoreType.DMA,
        pltpu.SemaphoreType.DMA,
        pltpu.SemaphoreType.DMA,
        pltpu.SemaphoreType.DMA,
    ],
    compiler_params=pltpu.CompilerParams(use_tc_tiling_on_sc=False),
)


def _tc_trig_body(a_ref, rad_ref, sin_ref, cos_ref):
    r = a_ref[...] * jnp.float32(math.pi / 180.0)
    rad_ref[...] = r
    sin_ref[...] = jnp.sin(r)
    cos_ref[...] = jnp.cos(r)


_TC_BLOCK = 1024

_tc_trig = pl.pallas_call(
    _tc_trig_body,
    grid=(_ROWS // _TC_BLOCK,),
    in_specs=[pl.BlockSpec((_TC_BLOCK, _COLS), lambda i: (i, 0))],
    out_specs=[pl.BlockSpec((_TC_BLOCK, _COLS), lambda i: (i, 0))] * 3,
    out_shape=[jax.ShapeDtypeStruct((_ROWS, _COLS), jnp.float32)] * 3,
)


def kernel(angles, table):
    radians, sin_enc, cos_enc = _tc_trig(angles)
    rows = _sc_gather(angles.reshape(_B // _IW, _IW), table)
    return radians, sin_enc, cos_enc, rows.reshape(_ROWS, _COLS, _EMBED)
